# baseline (speedup 1.0000x reference)
import jax
import jax.numpy as jnp
from jax.experimental import pallas as pl

_LS = 0.1
_PAD_IDX = 0
_EPS = 1e-05
_TRG_FACTOR = 1.0 - _LS
_NSPECIAL = 6


def kernel(pred, trg, biased_trg, biased_offset):
    b, s, v = pred.shape
    k = biased_trg.shape[-1]
    n = b * s
    base = _LS / (v - 2)

    pred2 = pred.reshape(n, v)
    tgt = trg.reshape(n, 1)
    pad = tgt == _PAD_IDX
    tval = (_TRG_FACTOR *
            (1.0 - biased_offset.reshape(-1)[(k - 1) * n:])).reshape(n, 1)
    tval = jnp.where(pad, _EPS, tval + _EPS)
    crow = jnp.where(pad, _EPS, base + _EPS)
    bt = biased_trg.reshape(n, k)
    no = jnp.where(pad, 0.0, (_TRG_FACTOR * biased_offset).reshape(n, k))
    pre_at_bt = jnp.where(bt == _PAD_IDX, _EPS,
                          jnp.where(bt == tgt, tval, crow))
    fbt = jnp.where(pad, _EPS, pre_at_bt + no)
    bd0 = jnp.zeros((n, 1), jnp.float32)
    for kk in [1, 0, 2, 3]:
        bd0 = jnp.where(bt[:, kk:kk + 1] == _PAD_IDX, no[:, kk:kk + 1], bd0)
    f0 = jnp.where(pad, _EPS, _EPS + bd0)

    icols = jnp.concatenate(
        [tgt, bt, jnp.full((n, 1), _PAD_IDX, jnp.int32),
         jnp.zeros((n, 2), jnp.int32)], axis=1)
    fvals = jnp.concatenate(
        [tval, fbt, f0, crow, jnp.zeros((n, 1), jnp.float32)], axis=1)

    block_rows = 512
    grid = (n // block_rows,)
    row_spec = lambda d: pl.BlockSpec((block_rows, d), lambda i: (i, 0))

    def body(pred_ref, icols_ref, fvals_ref, out_ref):
        rows, vocab = pred_ref.shape
        cols = jax.lax.broadcasted_iota(jnp.int32, (rows, vocab), 1)
        t = jnp.where(cols == icols_ref[:, 0:1], fvals_ref[:, 0:1],
                      fvals_ref[:, 6:7])
        for j in [2, 1, 3, 4, 5]:
            t = jnp.where(cols == icols_ref[:, j:j + 1],
                          fvals_ref[:, j:j + 1], t)
        out_ref[...] = t * (jnp.log(t) - pred_ref[...])

    return pl.pallas_call(
        body,
        grid=grid,
        in_specs=[
            row_spec(v),
            row_spec(8),
            row_spec(8),
        ],
        out_specs=row_spec(v),
        out_shape=jax.ShapeDtypeStruct((n, v), jnp.float32),
    )(pred2, icols, fvals)
